# baseline (speedup 1.0000x reference)
import jax
import jax.numpy as jnp
from jax import lax
from jax.experimental import pallas as pl
from jax.experimental.pallas import tpu as pltpu

_ALPHA = 0.01
_BG = 2.0
_SWAP = 2.0
_BIG = 1e30
_M = 512
_N = 512


def _soft_red_kernel(a_ref, b_ref, out_ref, g_ref, w_ref, t_ref):
    f32 = jnp.float32
    big = f32(_BIG)
    lane2d = lax.broadcasted_iota(jnp.int32, (_M, 128), 1)
    s_chunks = []
    for c in range(4):
        t_c = -lax.dot_general(
            b_ref[...], a_ref[128 * c:128 * (c + 1), :],
            dimension_numbers=(((1,), (1,)), ((), ())),
            preferred_element_type=f32)
        if c:
            t_c = jnp.concatenate(
                [t_c[_M - 128 * c:, :], t_c[:_M - 128 * c, :]], axis=0)
        for bit in range(7):
            s = 1 << bit
            rolled = jnp.concatenate([t_c[_M - s:, :], t_c[:_M - s, :]],
                                     axis=0)
            t_c = jnp.where((lane2d & s) != 0, rolled, t_c)
        s_chunks.append(t_c)

    rolls = [pltpu.roll(s_c, 1, axis=1) for s_c in s_chunks]
    lane0 = lane2d == 0
    w_chunks = []
    for c in range(4):
        sh_c = jnp.where(lane0, rolls[c - 1] if c else rolls[0], rolls[c])
        w_c = sh_c + s_chunks[c] + _SWAP
        if c == 0:
            w_c = jnp.where(lane0, big, w_c)
        w_chunks.append(w_c)

    snv = lax.broadcasted_iota(jnp.int32, (8, 128), 0)
    lnv = lax.broadcasted_iota(jnp.int32, (8, 128), 1)
    lmat = snv * 128 + lnv

    def repack(dst_ref):
        def grp(j, carry):
            base = pl.multiple_of(j * 8, 8)
            vs = [t_ref[c, pl.ds(base, 8), :] for c in range(4)]
            for t in range(8):
                acc = vs[0] * f32(0.0)
                for c in range(4):
                    rot = pltpu.roll(vs[c], (c - t) % 8, axis=0)
                    acc = jnp.where(snv == c, rot, acc)
                dst_ref[pl.ds(base + t, 1)] = acc.reshape(1, 8, 128)
            return carry
        lax.fori_loop(0, 64, grp, 0)

    for c in range(4):
        t_ref[c] = s_chunks[c]
    repack(g_ref)
    for c in range(4):
        t_ref[c] = w_chunks[c]
    repack(w_ref)

    zrow = jnp.reshape(g_ref[pl.ds(0, 1)], (8, 128)) * f32(0.0)
    e_init = zrow + big
    e1_0 = jnp.where(lmat == 0, f32(_BG), big) + zrow
    ln0 = lnv == 0
    ln01 = lnv <= 1

    def shift1(x, inj):
        a = pltpu.roll(x, 1, axis=1)
        a2 = pltpu.roll(a, 1, axis=0)
        r = jnp.where(ln0, a2, a)
        return jnp.where(lmat == 0, inj, r)

    def shift2(x, inj):
        a = pltpu.roll(x, 2, axis=1)
        a2 = pltpu.roll(a, 1, axis=0)
        r = jnp.where(ln01, a2, a)
        return jnp.where(lmat == 1, inj, r)

    def body(k, carry):
        e1, e2, e3, e4 = carry
        kf = k.astype(f32)
        kfb = kf * _BG
        r2 = (k - 2) & 511
        r3 = (k - 3) & 511
        g2 = jnp.reshape(g_ref[pl.ds(r2, 1)], (8, 128))
        w3 = jnp.reshape(w_ref[pl.ds(r3, 1)], (8, 128))

        sh_e1 = shift1(e1, kfb - _BG)
        sh_e2 = shift1(e2, kfb - 2.0 * _BG)
        inj4 = jnp.where(k >= 4, kfb - 4.0 * _BG, big)
        sh_e4 = shift2(e4, inj4)

        t1 = sh_e2 + g2
        t2 = sh_e1 + _BG
        t3 = e1 + _BG
        t4 = sh_e4 + w3
        m = jnp.minimum(jnp.minimum(t1, t2), jnp.minimum(t3, t4))
        ssum = (jnp.exp(-(t1 - m) / _ALPHA) + jnp.exp(-(t2 - m) / _ALPHA)
                + jnp.exp(-(t3 - m) / _ALPHA) + jnp.exp(-(t4 - m) / _ALPHA))
        d = -_ALPHA * jnp.log(ssum) + m
        e0 = jnp.where(lmat == k - 1, kfb, d)
        return (e0, e1, e2, e3)

    e_fin = lax.fori_loop(2, _M + _N + 1, body,
                          (e1_0, e_init, e_init, e_init),
                          unroll=4)[0]
    sel = jnp.where(lmat == _N - 1, e_fin, f32(0.0))
    out_ref[...] = jnp.sum(sel, axis=(0, 1), keepdims=True) * (1.0 / _M)


@jax.jit
def kernel(centers_a, centers_b):
    out = pl.pallas_call(
        _soft_red_kernel,
        out_shape=jax.ShapeDtypeStruct((1, 1), jnp.float32),
        scratch_shapes=[pltpu.VMEM((_M, 8, 128), jnp.float32),
                        pltpu.VMEM((_M, 8, 128), jnp.float32),
                        pltpu.VMEM((4, _M, 128), jnp.float32)],
    )(centers_a, centers_b)
    return out[0, 0]


# speedup vs baseline: 1.0236x; 1.0236x over previous
import jax
import jax.numpy as jnp
from jax import lax
from jax.experimental import pallas as pl
from jax.experimental.pallas import tpu as pltpu

_ALPHA = 0.01
_BG = 2.0
_SWAP = 2.0
_BIG = 1e30
_M = 512
_N = 512


def _soft_red_kernel(a_ref, b_ref, out_ref, g_ref, w_ref, t_ref):
    f32 = jnp.float32
    big = f32(_BIG)
    lane2d = lax.broadcasted_iota(jnp.int32, (_M, 128), 1)
    s_chunks = []
    for c in range(4):
        t_c = -lax.dot_general(
            b_ref[...], a_ref[128 * c:128 * (c + 1), :],
            dimension_numbers=(((1,), (1,)), ((), ())),
            preferred_element_type=f32)
        if c:
            t_c = jnp.concatenate(
                [t_c[_M - 128 * c:, :], t_c[:_M - 128 * c, :]], axis=0)
        for bit in range(7):
            s = 1 << bit
            rolled = jnp.concatenate([t_c[_M - s:, :], t_c[:_M - s, :]],
                                     axis=0)
            t_c = jnp.where((lane2d & s) != 0, rolled, t_c)
        s_chunks.append(t_c)

    rolls = [pltpu.roll(s_c, 1, axis=1) for s_c in s_chunks]
    lane0 = lane2d == 0
    w_chunks = []
    for c in range(4):
        sh_c = jnp.where(lane0, rolls[c - 1] if c else rolls[0], rolls[c])
        w_c = sh_c + s_chunks[c] + _SWAP
        if c == 0:
            w_c = jnp.where(lane0, big, w_c)
        w_chunks.append(w_c)

    snv = lax.broadcasted_iota(jnp.int32, (8, 128), 0)
    lnv = lax.broadcasted_iota(jnp.int32, (8, 128), 1)
    lmat = snv * 128 + lnv

    def repack(dst_ref):
        def grp(j, carry):
            base = pl.multiple_of(j * 8, 8)
            vs = [t_ref[c, pl.ds(base, 8), :] for c in range(4)]
            for t in range(8):
                acc = vs[0] * f32(0.0)
                for c in range(4):
                    rot = pltpu.roll(vs[c], (c - t) % 8, axis=0)
                    acc = jnp.where(snv == c, rot, acc)
                dst_ref[pl.ds(base + t, 1)] = acc.reshape(1, 8, 128)
            return carry
        lax.fori_loop(0, 64, grp, 0)

    for c in range(4):
        t_ref[c] = s_chunks[c]
    repack(g_ref)
    for c in range(4):
        t_ref[c] = w_chunks[c]
    repack(w_ref)

    zrow = jnp.reshape(g_ref[pl.ds(0, 1)], (8, 128)) * f32(0.0)
    e_init = zrow + big
    e1_0 = jnp.where(lmat == 0, f32(_BG), big) + zrow
    ln0 = lnv == 0
    ln01 = lnv <= 1

    _IEXP = 144.26950408889634
    _ALOG = 0.006931471805599453

    def shift1(x, inj):
        a = pltpu.roll(x, 1, axis=1)
        a2 = pltpu.roll(a, 1, axis=0)
        r = jnp.where(ln0, a2, a)
        return jnp.where(lmat == 0, inj, r)

    def body(k, carry):
        e1, h2, h13, h14 = carry
        kf = k.astype(f32)
        kfb = kf * _BG
        r2 = (k - 2) & 511
        r3 = (k - 3) & 511
        g2 = jnp.reshape(g_ref[pl.ds(r2, 1)], (8, 128))
        w3 = jnp.reshape(w_ref[pl.ds(r3, 1)], (8, 128))

        h1 = shift1(e1, kfb - _BG)
        sh_e4 = shift1(h14, big)

        t1 = h2 + g2
        t3 = e1 + _BG
        t4 = sh_e4 + w3
        mp = jnp.minimum(jnp.minimum(t1, t4), t3)
        t2 = h1 + _BG
        m = jnp.minimum(mp, t2)
        ssum = (jnp.exp2((m - t2) * _IEXP)
                + (jnp.exp2((m - t1) * _IEXP) + jnp.exp2((m - t3) * _IEXP)
                   + jnp.exp2((m - t4) * _IEXP)))
        d = m - _ALOG * jnp.log2(ssum)
        e0 = jnp.where(lmat == k - 1, kfb, d)
        return (e0, h1, h2, h13)

    h_1 = jnp.where(lmat == 0, f32(0.0), big) + zrow
    e_fin = lax.fori_loop(2, _M + _N + 1, body,
                          (e1_0, h_1, e_init, e_init),
                          unroll=4)[0]
    sel = jnp.where(lmat == _N - 1, e_fin, f32(0.0))
    out_ref[...] = jnp.sum(sel, axis=(0, 1), keepdims=True) * (1.0 / _M)


@jax.jit
def kernel(centers_a, centers_b):
    out = pl.pallas_call(
        _soft_red_kernel,
        out_shape=jax.ShapeDtypeStruct((1, 1), jnp.float32),
        scratch_shapes=[pltpu.VMEM((_M, 8, 128), jnp.float32),
                        pltpu.VMEM((_M, 8, 128), jnp.float32),
                        pltpu.VMEM((4, _M, 128), jnp.float32)],
    )(centers_a, centers_b)
    return out[0, 0]
